# Initial kernel scaffold; baseline (speedup 1.0000x reference)
import jax
import jax.numpy as jnp
from jax.experimental import pallas as pl


def kernel(x, edge_index, edge_attr, batch, params):
    raise NotImplementedError("write your pallas kernel here")


# speedup vs baseline: 1.0672x; 1.0672x over previous
import functools

import jax
import jax.numpy as jnp
from jax import lax
from jax.experimental import pallas as pl
from jax.experimental.pallas import tpu as pltpu
from jax.experimental.pallas import tpu_sc as plsc

ATOM_DIMS_K = [119, 4, 12, 12, 10, 6, 6, 2, 2]
NLAYER = 5
EMB = 300
N = 10000
E = 160000
G = 256

DP = 320
NW = 32
FW = DP // NW
CH = 128
NCHUNK = 1280
EPAD = NCHUNK * CH
SUP = 32
NSUP = NCHUNK // SUP
D2 = 640
NCODE = 64
NEG = -1e30
RB = 1000
NRB = N // RB

_f32 = jnp.float32
_i32 = jnp.int32


def _sc_compute_chunk(ib_dc, agg_v, rows, buf, j, ee_v):
    for g in range(8):
        dc = ib_dc[buf, j, pl.ds(g * 16, 16)]
        dstg = lax.shift_right_logical(dc, 6)
        codeg = jnp.bitwise_and(dc, 63)
        evec = lax.iota(_i32, 16) + (g * 16)
        dbase = dstg * FW
        for f in range(FW):
            fv = jnp.full((16,), f, _i32)
            hv = plsc.load_gather(rows, [evec, fv])
            ev = plsc.load_gather(ee_v, [codeg, fv])
            msg = jnp.maximum(hv + ev, 0.0)
            plsc.addupdate_scatter(agg_v, [dbase + f], msg)


def _sc_edge_body(tab_h, src_h, dc_h, ee_h, out_h,
                  ee_v, ib_src, ib_dc, r0, r1, agg_v, sg0, sg1, si):
    w = lax.axis_index("c") * 16 + lax.axis_index("s")

    @pl.loop(0, N * FW // 16, unroll=8)
    def _zero(i):
        agg_v[pl.ds(i * 16, 16)] = jnp.zeros((16,), _f32)

    pltpu.sync_copy(ee_h.at[w], ee_v)
    pltpu.sync_copy(src_h.at[pl.ds(0, SUP)], ib_src.at[0])
    pltpu.sync_copy(dc_h.at[pl.ds(0, SUP)], ib_dc.at[0])
    pltpu.async_copy(tab_h.at[w].at[ib_src.at[0].at[0]], r0, sg0)
    pltpu.async_copy(tab_h.at[w].at[ib_src.at[0].at[1]], r1, sg1)

    wait_src = tab_h.at[w].at[ib_src.at[0].at[0]]

    @pl.loop(0, NSUP)
    def _super(s):
        buf = jnp.bitwise_and(s, 1)
        nb = jnp.bitwise_and(s + 1, 1)

        @pl.when(s < NSUP - 1)
        def _pref():
            pltpu.async_copy(src_h.at[pl.ds((s + 1) * SUP, SUP)],
                             ib_src.at[nb], si)
            pltpu.async_copy(dc_h.at[pl.ds((s + 1) * SUP, SUP)],
                             ib_dc.at[nb], si)

        @pl.loop(0, SUP - 2, step=2)
        def _main(j0):
            for b, (rr, sg) in enumerate(((r0, sg0), (r1, sg1))):
                j = j0 + b
                pltpu.make_async_copy(wait_src, rr, sg).wait()
                _sc_compute_chunk(ib_dc, agg_v, rr, buf, j, ee_v)
                pltpu.async_copy(tab_h.at[w].at[ib_src.at[buf].at[j + 2]],
                                 rr, sg)

        @pl.when(s < NSUP - 1)
        def _wait_idx():
            pltpu.make_async_copy(src_h.at[pl.ds(0, SUP)], ib_src.at[nb],
                                  si).wait()
            pltpu.make_async_copy(dc_h.at[pl.ds(0, SUP)], ib_dc.at[nb],
                                  si).wait()

        for b, (rr, sg) in enumerate(((r0, sg0), (r1, sg1))):
            j = SUP - 2 + b
            pltpu.make_async_copy(wait_src, rr, sg).wait()
            _sc_compute_chunk(ib_dc, agg_v, rr, buf, j, ee_v)
            pltpu.async_copy(tab_h.at[w].at[ib_src.at[nb].at[b]], rr, sg)

    pltpu.make_async_copy(wait_src, r0, sg0).wait()
    pltpu.make_async_copy(wait_src, r1, sg1).wait()

    pltpu.sync_copy(agg_v, out_h.at[w])


def _sc_edge(tab, src_idx, dc_idx, ee_tab):
    mesh = plsc.VectorSubcoreMesh(core_axis_name="c", subcore_axis_name="s")
    return pl.kernel(
        _sc_edge_body,
        out_type=jax.ShapeDtypeStruct((NW, N * FW), _f32),
        mesh=mesh,
        compiler_params=pltpu.CompilerParams(
            needs_layout_passes=False, use_tc_tiling_on_sc=False),
        scratch_types=[
            pltpu.VMEM((NCODE, FW), _f32),
            pltpu.VMEM((2, SUP, CH), _i32),
            pltpu.VMEM((2, SUP, CH), _i32),
            pltpu.VMEM((CH, FW), _f32),
            pltpu.VMEM((CH, FW), _f32),
            pltpu.VMEM((N * FW,), _f32),
            pltpu.SemaphoreType.DMA,
            pltpu.SemaphoreType.DMA,
            pltpu.SemaphoreType.DMA,
        ],
    )(tab, src_idx, dc_idx, ee_tab)


def _rb(d):
    return pl.BlockSpec((RB, d), lambda i: (i, 0))


def _full(*shape):
    return pl.BlockSpec(shape, lambda i: tuple(0 for _ in shape))


def _kh_body(x_ref, atab_ref, vnrow_ref, h_ref):
    x = x_ref[...]
    h = jnp.broadcast_to(vnrow_ref[...], (RB, DP))
    off = 0
    for i, d in enumerate(ATOM_DIMS_K):
        io = lax.broadcasted_iota(_i32, (RB, d), 1)
        oh = (io == x[:, i:i + 1]).astype(_f32)
        h = h + jnp.dot(oh, atab_ref[off:off + d],
                        preferred_element_type=_f32)
        off += d
    h_ref[...] = h


def _kob_body(b_ref, ob_ref):
    gio = lax.broadcasted_iota(_i32, (RB, G), 1)
    ob_ref[...] = (b_ref[...] == gio).astype(_f32)


def _k0(x, batch2d, atab, vnrow):
    h = pl.pallas_call(
        _kh_body,
        grid=(NRB,),
        in_specs=[_rb(16), _full(173, DP), _full(1, DP)],
        out_specs=_rb(DP),
        out_shape=jax.ShapeDtypeStruct((N, DP), _f32))(x, atab, vnrow)
    ob = pl.pallas_call(
        _kob_body,
        grid=(NRB,),
        in_specs=[_rb(1)],
        out_specs=_rb(G),
        out_shape=jax.ShapeDtypeStruct((N, G), _f32))(batch2d)
    return h, ob


def _z_body(hc_ref, agg_ref, eps_ref, z_ref):
    z_ref[...] = (1.0 + eps_ref[0, 0]) * hc_ref[...] + agg_ref[...]


def _z_call(hc, agg, eps):
    return pl.pallas_call(
        _z_body, grid=(NRB,),
        in_specs=[_rb(DP), _rb(DP), _full(1, 1)],
        out_specs=_rb(DP),
        out_shape=jax.ShapeDtypeStruct((N, DP), _f32))(hc, agg, eps)


def _mm_body(a_ref, w_ref, b_ref, y_ref):
    y_ref[...] = jnp.dot(a_ref[...], w_ref[...],
                         preferred_element_type=_f32) + b_ref[...]


def _mm_call(a, w, b):
    k, m = w.shape
    return pl.pallas_call(
        _mm_body, grid=(NRB,),
        in_specs=[_rb(k), _full(k, m), _full(1, m)],
        out_specs=_rb(m),
        out_shape=jax.ShapeDtypeStruct((N, m), _f32))(a, w, b)


def _stats_body(y_ref, s_ref):
    i = pl.program_id(0)

    @pl.when(i == 0)
    def _init():
        s_ref[...] = jnp.zeros_like(s_ref)

    y = y_ref[...]
    s_ref[0:1] += jnp.sum(y, axis=0, keepdims=True) * (1.0 / N)
    s_ref[1:2] += jnp.sum(y * y, axis=0, keepdims=True) * (1.0 / N)


def _stats_call(y):
    d = y.shape[1]
    return pl.pallas_call(
        _stats_body, grid=(NRB,),
        in_specs=[_rb(d)],
        out_specs=_full(2, d),
        out_shape=jax.ShapeDtypeStruct((2, d), _f32))(y)


def _bn_from_stats(y, s, g, b):
    mu = s[0:1]
    var = s[1:2] - mu * mu
    return g * (y - mu) * lax.rsqrt(var + 1e-5) + b


def _norm_relu_body(y_ref, s_ref, g_ref, b_ref, o_ref):
    o_ref[...] = jnp.maximum(
        _bn_from_stats(y_ref[...], s_ref[...], g_ref[...], b_ref[...]), 0.0)


def _norm_relu_call(y, s, g, b):
    d = y.shape[1]
    return pl.pallas_call(
        _norm_relu_body, grid=(NRB,),
        in_specs=[_rb(d), _full(2, d), _full(1, d), _full(1, d)],
        out_specs=_rb(d),
        out_shape=jax.ShapeDtypeStruct((N, d), _f32))(y, s, g, b)


def _vn_body(hc_ref, ob_ref, vn_ref, v1_ref, vb1_ref, vg1_ref, vt1_ref,
             v2_ref, vb2_ref, vg2_ref, vt2_ref, vo_ref, acc):
    i = pl.program_id(0)

    @pl.when(i == 0)
    def _init():
        acc[...] = jnp.zeros_like(acc)

    acc[...] += lax.dot_general(ob_ref[...], hc_ref[...],
                                (((0,), (0,)), ((), ())),
                                preferred_element_type=_f32)

    @pl.when(i == NRB - 1)
    def _fin():
        vtmp = acc[...] + vn_ref[...]
        v = jnp.dot(vtmp, v1_ref[...], preferred_element_type=_f32) + vb1_ref[...]
        mu = jnp.mean(v, axis=0, keepdims=True)
        dv = v - mu
        var = jnp.mean(dv * dv, axis=0, keepdims=True)
        v = jnp.maximum(vg1_ref[...] * dv * lax.rsqrt(var + 1e-5)
                        + vt1_ref[...], 0.0)
        v = jnp.dot(v, v2_ref[...], preferred_element_type=_f32) + vb2_ref[...]
        mu = jnp.mean(v, axis=0, keepdims=True)
        dv = v - mu
        var = jnp.mean(dv * dv, axis=0, keepdims=True)
        vo_ref[...] = jnp.maximum(
            vg2_ref[...] * dv * lax.rsqrt(var + 1e-5) + vt2_ref[...], 0.0)


def _vn_call(hc, ob, vn, vw):
    return pl.pallas_call(
        _vn_body, grid=(NRB,),
        in_specs=[_rb(DP), _rb(G), _full(G, DP),
                  _full(DP, D2), _full(1, D2), _full(1, D2), _full(1, D2),
                  _full(D2, DP), _full(1, DP), _full(1, DP), _full(1, DP)],
        out_specs=_full(G, DP),
        out_shape=jax.ShapeDtypeStruct((G, DP), _f32),
        scratch_shapes=[pltpu.VMEM((G, DP), _f32)])(hc, ob, vn, *vw)


def _fin_body(y_ref, s_ref, g_ref, b_ref, ob_ref, vnn_ref, hn_ref):
    out = jnp.maximum(
        _bn_from_stats(y_ref[...], s_ref[...], g_ref[...], b_ref[...]), 0.0)
    hn_ref[...] = out + jnp.dot(ob_ref[...], vnn_ref[...],
                                preferred_element_type=_f32)


def _fin_call(y2, s2, bg, bb, ob, vnn):
    return pl.pallas_call(
        _fin_body, grid=(NRB,),
        in_specs=[_rb(DP), _full(2, DP), _full(1, DP), _full(1, DP),
                  _rb(G), _full(G, DP)],
        out_specs=_rb(DP),
        out_shape=jax.ShapeDtypeStruct((N, DP), _f32))(y2, s2, bg, bb, ob, vnn)


def _pool_body(y_ref, s_ref, g_ref, b_ref, ob_ref, p_ref, acc, cacc):
    i = pl.program_id(0)

    @pl.when(i == 0)
    def _init():
        acc[...] = jnp.zeros_like(acc)
        cacc[...] = jnp.zeros_like(cacc)

    out = _bn_from_stats(y_ref[...], s_ref[...], g_ref[...], b_ref[...])
    ob = ob_ref[...]
    acc[...] += lax.dot_general(ob, out, (((0,), (0,)), ((), ())),
                                preferred_element_type=_f32)
    cacc[...] += jnp.sum(ob, axis=0, keepdims=True)

    @pl.when(i == NRB - 1)
    def _fin():
        p_ref[...] = acc[...] / jnp.maximum(cacc[...], 1.0).T


def _pool_call(y2, s2, bg, bb, ob):
    return pl.pallas_call(
        _pool_body, grid=(NRB,),
        in_specs=[_rb(DP), _full(2, DP), _full(1, DP), _full(1, DP), _rb(G)],
        out_specs=_full(G, DP),
        out_shape=jax.ShapeDtypeStruct((G, DP), _f32),
        scratch_shapes=[pltpu.VMEM((G, DP), _f32),
                        pltpu.VMEM((1, G), _f32)])(y2, s2, bg, bb, ob)


def _k_layer(l, hc, agg, vn, ob, wl):
    (eps, w1, b1, g1, t1, w2, b2, bg, bb), vw = wl
    z = _z_call(hc, agg, eps)
    y1 = _mm_call(z, w1, b1)
    s1 = _stats_call(y1)
    a1 = _norm_relu_call(y1, s1, g1, t1)
    y2 = _mm_call(a1, w2, b2)
    s2 = _stats_call(y2)
    if l < NLAYER - 1:
        vnn = _vn_call(hc, ob, vn, vw)
        hn = _fin_call(y2, s2, bg, bb, ob, vnn)
        return hn, vnn
    return _pool_call(y2, s2, bg, bb, ob), vn


def _padw(a, r, c):
    return jnp.pad(a, ((0, r - a.shape[0]), (0, c - a.shape[1])))


def _padv(a, c):
    return jnp.pad(a, (0, c - a.shape[0])).reshape(1, c)


def _split_nodes(h):
    return h.reshape(N, NW, FW).transpose(1, 0, 2)


def _unsplit_nodes(a):
    return a.reshape(NW, N, FW).transpose(1, 0, 2).reshape(N, DP)


def kernel(x, edge_index, edge_attr, batch, params):
    src = edge_index[0].astype(_i32)
    dst = edge_index[1].astype(_i32)
    code = (edge_attr[:, 0] + 5 * edge_attr[:, 1] +
            30 * edge_attr[:, 2]).astype(_i32)
    src_idx = jnp.pad(src, (0, EPAD - E)).reshape(NCHUNK, CH)
    dc_idx = jnp.pad(dst * 64 + code, (0, EPAD - E),
                     constant_values=63).reshape(NCHUNK, CH)

    atab = jnp.concatenate(params["atom_emb"], axis=0)
    atab = jnp.pad(atab, ((0, 0), (0, DP - EMB)))
    vnrow = _padv(params["vn_emb"][0], DP)
    batch2d = batch.astype(_i32).reshape(N, 1)
    xpad = jnp.pad(x.astype(_i32), ((0, 0), (0, 16 - x.shape[1])))

    h0, ob = _k0(xpad, batch2d, atab, vnrow)

    layer_w = []
    ee_tabs = []
    for l in range(NLAYER):
        cp = params["convs"][l]
        bn = params["bns"][l]
        b0, b1, b2 = cp["bond_emb"]
        ee = (b0[:, None, None, :] + b1[None, :, None, :] +
              b2[None, None, :, :])
        ee = ee.transpose(2, 1, 0, 3).reshape(60, EMB)
        ee = jnp.pad(ee, ((0, 0), (0, DP - EMB)))
        ee = jnp.concatenate([ee, jnp.full((NCODE - 60, DP), NEG, _f32)], 0)
        ee_tabs.append(ee.reshape(NCODE, NW, FW).transpose(1, 0, 2))
        conv_w = (
            cp["eps"].reshape(1, 1),
            _padw(cp["W1"], DP, D2), _padv(cp["b1"], D2),
            _padv(cp["g1"], D2), _padv(cp["bt1"], D2),
            _padw(cp["W2"], D2, DP), _padv(cp["b2"], DP),
            _padv(bn["g"], DP), _padv(bn["b"], DP),
        )
        if l < NLAYER - 1:
            vp = params["vn_mlps"][l]
            vw = (
                _padw(vp["W1"], DP, D2), _padv(vp["b1"], D2),
                _padv(vp["g1"], D2), _padv(vp["bt1"], D2),
                _padw(vp["W2"], D2, DP), _padv(vp["b2"], DP),
                _padv(vp["g2"], DP), _padv(vp["bt2"], DP),
            )
        else:
            vw = None
        layer_w.append((conv_w, vw))

    hc = h0
    vn = jnp.broadcast_to(vnrow, (G, DP))
    for l in range(NLAYER):
        tab = _split_nodes(hc)
        agg = _unsplit_nodes(_sc_edge(tab, src_idx, dc_idx, ee_tabs[l]))
        hc, vn = _k_layer(l, hc, agg, vn, ob, layer_w[l])

    return hc[0:G, 0:EMB]


# speedup vs baseline: 1.5287x; 1.4324x over previous
import functools

import jax
import jax.numpy as jnp
from jax import lax
from jax.experimental import pallas as pl
from jax.experimental.pallas import tpu as pltpu
from jax.experimental.pallas import tpu_sc as plsc

ATOM_DIMS_K = [119, 4, 12, 12, 10, 6, 6, 2, 2]
NLAYER = 5
EMB = 300
N = 10000
E = 160000
G = 256

DP = 320
NW = 32
FW = DP // NW
CH = 128
NCHUNK = 1280
EPAD = NCHUNK * CH
SUP = 32
NSUP = NCHUNK // SUP
D2 = 640
NCODE = 64
NEG = -1e30
RB = 1000
NRB = N // RB

_f32 = jnp.float32
_i32 = jnp.int32


def _sc_compute_chunk(ib_dc, agg_v, rows, buf, j, ee_v):
    def _ld(g):
        dc = ib_dc[buf, j, pl.ds(g * 16, 16)]
        dstg = lax.shift_right_logical(dc, 6)
        codeg = jnp.bitwise_and(dc, 63)
        evec = lax.iota(_i32, 16) + g * 16
        msgs = []
        for f in range(FW):
            fv = jnp.full((16,), f, _i32)
            hv = plsc.load_gather(rows, [evec, fv])
            ev = plsc.load_gather(ee_v, [codeg, fv])
            msgs.append(jnp.maximum(hv + ev, 0.0))
        return dstg * FW, msgs

    cur = _ld(0)
    for g in range(8):
        nxt = _ld(g + 1) if g < 7 else None
        dbase, msgs = cur
        for f in range(FW):
            plsc.addupdate_scatter(agg_v, [dbase + f], msgs[f])
        cur = nxt


def _sc_edge_body(tab_h, src_h, dc_h, ee_h, out_h,
                  ee_v, ib_src, ib_dc, r0, r1, agg_v, sg0, sg1, si):
    w = lax.axis_index("c") * 16 + lax.axis_index("s")

    @pl.loop(0, N * FW // 16, unroll=8)
    def _zero(i):
        agg_v[pl.ds(i * 16, 16)] = jnp.zeros((16,), _f32)

    pltpu.sync_copy(ee_h.at[w], ee_v)
    pltpu.sync_copy(src_h.at[pl.ds(0, SUP)], ib_src.at[0])
    pltpu.sync_copy(dc_h.at[pl.ds(0, SUP)], ib_dc.at[0])
    pltpu.async_copy(tab_h.at[w].at[ib_src.at[0].at[0]], r0, sg0)
    pltpu.async_copy(tab_h.at[w].at[ib_src.at[0].at[1]], r1, sg1)

    wait_src = tab_h.at[w].at[ib_src.at[0].at[0]]

    @pl.loop(0, NSUP)
    def _super(s):
        buf = jnp.bitwise_and(s, 1)
        nb = jnp.bitwise_and(s + 1, 1)

        @pl.when(s < NSUP - 1)
        def _pref():
            pltpu.async_copy(src_h.at[pl.ds((s + 1) * SUP, SUP)],
                             ib_src.at[nb], si)
            pltpu.async_copy(dc_h.at[pl.ds((s + 1) * SUP, SUP)],
                             ib_dc.at[nb], si)

        @pl.loop(0, SUP - 2, step=2)
        def _main(j0):
            for b, (rr, sg) in enumerate(((r0, sg0), (r1, sg1))):
                j = j0 + b
                pltpu.make_async_copy(wait_src, rr, sg).wait()
                _sc_compute_chunk(ib_dc, agg_v, rr, buf, j, ee_v)
                pltpu.async_copy(tab_h.at[w].at[ib_src.at[buf].at[j + 2]],
                                 rr, sg)

        @pl.when(s < NSUP - 1)
        def _wait_idx():
            pltpu.make_async_copy(src_h.at[pl.ds(0, SUP)], ib_src.at[nb],
                                  si).wait()
            pltpu.make_async_copy(dc_h.at[pl.ds(0, SUP)], ib_dc.at[nb],
                                  si).wait()

        for b, (rr, sg) in enumerate(((r0, sg0), (r1, sg1))):
            j = SUP - 2 + b
            pltpu.make_async_copy(wait_src, rr, sg).wait()
            _sc_compute_chunk(ib_dc, agg_v, rr, buf, j, ee_v)
            pltpu.async_copy(tab_h.at[w].at[ib_src.at[nb].at[b]], rr, sg)

    pltpu.make_async_copy(wait_src, r0, sg0).wait()
    pltpu.make_async_copy(wait_src, r1, sg1).wait()

    pltpu.sync_copy(agg_v, out_h.at[w])


def _sc_edge(tab, src_idx, dc_idx, ee_tab):
    mesh = plsc.VectorSubcoreMesh(core_axis_name="c", subcore_axis_name="s")
    return pl.kernel(
        _sc_edge_body,
        out_type=jax.ShapeDtypeStruct((NW, N * FW), _f32),
        mesh=mesh,
        compiler_params=pltpu.CompilerParams(
            needs_layout_passes=False, use_tc_tiling_on_sc=False),
        scratch_types=[
            pltpu.VMEM((NCODE, FW), _f32),
            pltpu.VMEM((2, SUP, CH), _i32),
            pltpu.VMEM((2, SUP, CH), _i32),
            pltpu.VMEM((CH, FW), _f32),
            pltpu.VMEM((CH, FW), _f32),
            pltpu.VMEM((N * FW,), _f32),
            pltpu.SemaphoreType.DMA,
            pltpu.SemaphoreType.DMA,
            pltpu.SemaphoreType.DMA,
        ],
    )(tab, src_idx, dc_idx, ee_tab)


def _rb(d):
    return pl.BlockSpec((RB, d), lambda i: (i, 0))


def _full(*shape):
    return pl.BlockSpec(shape, lambda i: tuple(0 for _ in shape))


def _kh_body(x_ref, atab_ref, vnrow_ref, h_ref):
    x = x_ref[...]
    h = jnp.broadcast_to(vnrow_ref[...], (RB, DP))
    off = 0
    for i, d in enumerate(ATOM_DIMS_K):
        io = lax.broadcasted_iota(_i32, (RB, d), 1)
        oh = (io == x[:, i:i + 1]).astype(_f32)
        h = h + jnp.dot(oh, atab_ref[off:off + d],
                        preferred_element_type=_f32)
        off += d
    h_ref[...] = h


def _kob_body(b_ref, ob_ref):
    gio = lax.broadcasted_iota(_i32, (RB, G), 1)
    ob_ref[...] = (b_ref[...] == gio).astype(_f32)


def _k0(x, batch2d, atab, vnrow):
    h = pl.pallas_call(
        _kh_body,
        grid=(NRB,),
        in_specs=[_rb(16), _full(173, DP), _full(1, DP)],
        out_specs=_rb(DP),
        out_shape=jax.ShapeDtypeStruct((N, DP), _f32))(x, atab, vnrow)
    ob = pl.pallas_call(
        _kob_body,
        grid=(NRB,),
        in_specs=[_rb(1)],
        out_specs=_rb(G),
        out_shape=jax.ShapeDtypeStruct((N, G), _f32))(batch2d)
    return h, ob


def _z_body(hc_ref, agg_ref, eps_ref, z_ref):
    z_ref[...] = (1.0 + eps_ref[0, 0]) * hc_ref[...] + agg_ref[...]


def _z_call(hc, agg, eps):
    return pl.pallas_call(
        _z_body, grid=(NRB,),
        in_specs=[_rb(DP), _rb(DP), _full(1, 1)],
        out_specs=_rb(DP),
        out_shape=jax.ShapeDtypeStruct((N, DP), _f32))(hc, agg, eps)


def _mm_body(a_ref, w_ref, b_ref, y_ref):
    y_ref[...] = jnp.dot(a_ref[...], w_ref[...],
                         preferred_element_type=_f32) + b_ref[...]


def _mm_call(a, w, b):
    k, m = w.shape
    return pl.pallas_call(
        _mm_body, grid=(NRB,),
        in_specs=[_rb(k), _full(k, m), _full(1, m)],
        out_specs=_rb(m),
        out_shape=jax.ShapeDtypeStruct((N, m), _f32))(a, w, b)


def _stats_body(y_ref, s_ref):
    i = pl.program_id(0)

    @pl.when(i == 0)
    def _init():
        s_ref[...] = jnp.zeros_like(s_ref)

    y = y_ref[...]
    s_ref[0:1] += jnp.sum(y, axis=0, keepdims=True) * (1.0 / N)
    s_ref[1:2] += jnp.sum(y * y, axis=0, keepdims=True) * (1.0 / N)


def _stats_call(y):
    d = y.shape[1]
    return pl.pallas_call(
        _stats_body, grid=(NRB,),
        in_specs=[_rb(d)],
        out_specs=_full(2, d),
        out_shape=jax.ShapeDtypeStruct((2, d), _f32))(y)


def _bn_from_stats(y, s, g, b):
    mu = s[0:1]
    var = s[1:2] - mu * mu
    return g * (y - mu) * lax.rsqrt(var + 1e-5) + b


def _norm_relu_body(y_ref, s_ref, g_ref, b_ref, o_ref):
    o_ref[...] = jnp.maximum(
        _bn_from_stats(y_ref[...], s_ref[...], g_ref[...], b_ref[...]), 0.0)


def _norm_relu_call(y, s, g, b):
    d = y.shape[1]
    return pl.pallas_call(
        _norm_relu_body, grid=(NRB,),
        in_specs=[_rb(d), _full(2, d), _full(1, d), _full(1, d)],
        out_specs=_rb(d),
        out_shape=jax.ShapeDtypeStruct((N, d), _f32))(y, s, g, b)


def _vn_body(hc_ref, ob_ref, vn_ref, v1_ref, vb1_ref, vg1_ref, vt1_ref,
             v2_ref, vb2_ref, vg2_ref, vt2_ref, vo_ref, acc):
    i = pl.program_id(0)

    @pl.when(i == 0)
    def _init():
        acc[...] = jnp.zeros_like(acc)

    acc[...] += lax.dot_general(ob_ref[...], hc_ref[...],
                                (((0,), (0,)), ((), ())),
                                preferred_element_type=_f32)

    @pl.when(i == NRB - 1)
    def _fin():
        vtmp = acc[...] + vn_ref[...]
        v = jnp.dot(vtmp, v1_ref[...], preferred_element_type=_f32) + vb1_ref[...]
        mu = jnp.mean(v, axis=0, keepdims=True)
        dv = v - mu
        var = jnp.mean(dv * dv, axis=0, keepdims=True)
        v = jnp.maximum(vg1_ref[...] * dv * lax.rsqrt(var + 1e-5)
                        + vt1_ref[...], 0.0)
        v = jnp.dot(v, v2_ref[...], preferred_element_type=_f32) + vb2_ref[...]
        mu = jnp.mean(v, axis=0, keepdims=True)
        dv = v - mu
        var = jnp.mean(dv * dv, axis=0, keepdims=True)
        vo_ref[...] = jnp.maximum(
            vg2_ref[...] * dv * lax.rsqrt(var + 1e-5) + vt2_ref[...], 0.0)


def _vn_call(hc, ob, vn, vw):
    return pl.pallas_call(
        _vn_body, grid=(NRB,),
        in_specs=[_rb(DP), _rb(G), _full(G, DP),
                  _full(DP, D2), _full(1, D2), _full(1, D2), _full(1, D2),
                  _full(D2, DP), _full(1, DP), _full(1, DP), _full(1, DP)],
        out_specs=_full(G, DP),
        out_shape=jax.ShapeDtypeStruct((G, DP), _f32),
        scratch_shapes=[pltpu.VMEM((G, DP), _f32)])(hc, ob, vn, *vw)


def _fin_body(y_ref, s_ref, g_ref, b_ref, ob_ref, vnn_ref, hn_ref):
    out = jnp.maximum(
        _bn_from_stats(y_ref[...], s_ref[...], g_ref[...], b_ref[...]), 0.0)
    hn_ref[...] = out + jnp.dot(ob_ref[...], vnn_ref[...],
                                preferred_element_type=_f32)


def _fin_call(y2, s2, bg, bb, ob, vnn):
    return pl.pallas_call(
        _fin_body, grid=(NRB,),
        in_specs=[_rb(DP), _full(2, DP), _full(1, DP), _full(1, DP),
                  _rb(G), _full(G, DP)],
        out_specs=_rb(DP),
        out_shape=jax.ShapeDtypeStruct((N, DP), _f32))(y2, s2, bg, bb, ob, vnn)


def _pool_body(y_ref, s_ref, g_ref, b_ref, ob_ref, p_ref, acc, cacc):
    i = pl.program_id(0)

    @pl.when(i == 0)
    def _init():
        acc[...] = jnp.zeros_like(acc)
        cacc[...] = jnp.zeros_like(cacc)

    out = _bn_from_stats(y_ref[...], s_ref[...], g_ref[...], b_ref[...])
    ob = ob_ref[...]
    acc[...] += lax.dot_general(ob, out, (((0,), (0,)), ((), ())),
                                preferred_element_type=_f32)
    cacc[...] += jnp.sum(ob, axis=0, keepdims=True)

    @pl.when(i == NRB - 1)
    def _fin():
        p_ref[...] = acc[...] / jnp.maximum(cacc[...], 1.0).T


def _pool_call(y2, s2, bg, bb, ob):
    return pl.pallas_call(
        _pool_body, grid=(NRB,),
        in_specs=[_rb(DP), _full(2, DP), _full(1, DP), _full(1, DP), _rb(G)],
        out_specs=_full(G, DP),
        out_shape=jax.ShapeDtypeStruct((G, DP), _f32),
        scratch_shapes=[pltpu.VMEM((G, DP), _f32),
                        pltpu.VMEM((1, G), _f32)])(y2, s2, bg, bb, ob)


def _k_layer(l, hc, agg, vn, ob, wl):
    (eps, w1, b1, g1, t1, w2, b2, bg, bb), vw = wl
    z = _z_call(hc, agg, eps)
    y1 = _mm_call(z, w1, b1)
    s1 = _stats_call(y1)
    a1 = _norm_relu_call(y1, s1, g1, t1)
    y2 = _mm_call(a1, w2, b2)
    s2 = _stats_call(y2)
    if l < NLAYER - 1:
        vnn = _vn_call(hc, ob, vn, vw)
        hn = _fin_call(y2, s2, bg, bb, ob, vnn)
        return hn, vnn
    return _pool_call(y2, s2, bg, bb, ob), vn


def _padw(a, r, c):
    return jnp.pad(a, ((0, r - a.shape[0]), (0, c - a.shape[1])))


def _padv(a, c):
    return jnp.pad(a, (0, c - a.shape[0])).reshape(1, c)


def _split_nodes(h):
    return h.reshape(N, NW, FW).transpose(1, 0, 2)


def _unsplit_nodes(a):
    return a.reshape(NW, N, FW).transpose(1, 0, 2).reshape(N, DP)


def kernel(x, edge_index, edge_attr, batch, params):
    src = edge_index[0].astype(_i32)
    dst = edge_index[1].astype(_i32)
    code = (edge_attr[:, 0] + 5 * edge_attr[:, 1] +
            30 * edge_attr[:, 2]).astype(_i32)
    src_idx = jnp.pad(src, (0, EPAD - E)).reshape(NCHUNK, CH)
    dc_idx = jnp.pad(dst * 64 + code, (0, EPAD - E),
                     constant_values=63).reshape(NCHUNK, CH)

    atab = jnp.concatenate(params["atom_emb"], axis=0)
    atab = jnp.pad(atab, ((0, 0), (0, DP - EMB)))
    vnrow = _padv(params["vn_emb"][0], DP)
    batch2d = batch.astype(_i32).reshape(N, 1)
    xpad = jnp.pad(x.astype(_i32), ((0, 0), (0, 16 - x.shape[1])))

    h0, ob = _k0(xpad, batch2d, atab, vnrow)

    layer_w = []
    ee_tabs = []
    for l in range(NLAYER):
        cp = params["convs"][l]
        bn = params["bns"][l]
        b0, b1, b2 = cp["bond_emb"]
        ee = (b0[:, None, None, :] + b1[None, :, None, :] +
              b2[None, None, :, :])
        ee = ee.transpose(2, 1, 0, 3).reshape(60, EMB)
        ee = jnp.pad(ee, ((0, 0), (0, DP - EMB)))
        ee = jnp.concatenate([ee, jnp.full((NCODE - 60, DP), NEG, _f32)], 0)
        ee_tabs.append(ee.reshape(NCODE, NW, FW).transpose(1, 0, 2))
        conv_w = (
            cp["eps"].reshape(1, 1),
            _padw(cp["W1"], DP, D2), _padv(cp["b1"], D2),
            _padv(cp["g1"], D2), _padv(cp["bt1"], D2),
            _padw(cp["W2"], D2, DP), _padv(cp["b2"], DP),
            _padv(bn["g"], DP), _padv(bn["b"], DP),
        )
        if l < NLAYER - 1:
            vp = params["vn_mlps"][l]
            vw = (
                _padw(vp["W1"], DP, D2), _padv(vp["b1"], D2),
                _padv(vp["g1"], D2), _padv(vp["bt1"], D2),
                _padw(vp["W2"], D2, DP), _padv(vp["b2"], DP),
                _padv(vp["g2"], DP), _padv(vp["bt2"], DP),
            )
        else:
            vw = None
        layer_w.append((conv_w, vw))

    hc = h0
    vn = jnp.broadcast_to(vnrow, (G, DP))
    for l in range(NLAYER):
        tab = _split_nodes(hc)
        agg = _unsplit_nodes(_sc_edge(tab, src_idx, dc_idx, ee_tabs[l]))
        hc, vn = _k_layer(l, hc, agg, vn, ob, layer_w[l])

    return hc[0:G, 0:EMB]
